# baseline (speedup 1.0000x reference)
import jax
import jax.numpy as jnp
from jax.experimental import pallas as pl

NUM_CLASSES = 1000
BATCH = 16384
PR = 8000


def _body(x_ref, out_ref):
    i = pl.program_id(0)
    part = jnp.sum(x_ref[...]) * (1.0 / BATCH)

    @pl.when(i == 0)
    def _():
        out_ref[...] = jnp.zeros((1, 1), jnp.float32)

    out_ref[...] += jnp.reshape(part, (1, 1))


@jax.jit
def kernel(logits, targets):
    x128 = logits.reshape(BATCH * NUM_CLASSES // 128, 128)
    out = pl.pallas_call(
        _body,
        grid=(BATCH * NUM_CLASSES // 128 // PR,),
        in_specs=[pl.BlockSpec((PR, 128), lambda i: (i, 0))],
        out_specs=pl.BlockSpec((1, 1), lambda i: (0, 0)),
        out_shape=jax.ShapeDtypeStruct((1, 1), jnp.float32),
    )(x128)
    return out[0, 0]


# speedup vs baseline: 1.9377x; 1.9377x over previous
import jax
import jax.numpy as jnp
from jax import lax
from jax.experimental import pallas as pl
from jax.experimental.pallas import tpu as pltpu

NUM_CLASSES = 1000
EPS = 0.1
BATCH = 16384
A = EPS / (NUM_CLASSES - 1)
B_COEF = 1.0 - EPS - A

BR = 2048


def _loss_body(t_ref, x_hbm, out_ref, buf, sems):
    i = pl.program_id(0)
    n = pl.num_programs(0)
    slot = i % 2

    @pl.when(i == 0)
    def _():
        pltpu.make_async_copy(
            x_hbm.at[pl.ds(0, BR)], buf.at[0], sems.at[0]
        ).start()

    @pl.when(i + 1 < n)
    def _():
        pltpu.make_async_copy(
            x_hbm.at[pl.ds((i + 1) * BR, BR)],
            buf.at[(i + 1) % 2],
            sems.at[(i + 1) % 2],
        ).start()

    pltpu.make_async_copy(
        x_hbm.at[pl.ds(i * BR, BR)], buf.at[slot], sems.at[slot]
    ).wait()
    x = buf[slot]
    t = t_ref[0, 0, :]
    m = jnp.max(x, axis=1, keepdims=True)
    s = jnp.sum(jnp.exp(x - m), axis=1)
    lse = jnp.log(s) + m[:, 0]
    row_sum = jnp.sum(x, axis=1)
    col = lax.broadcasted_iota(jnp.int32, x.shape, 1)
    tgt = jnp.sum(jnp.where(col == t[:, None], x, 0.0), axis=1)
    part = jnp.sum(lse - A * row_sum - B_COEF * tgt) * (1.0 / BATCH)

    @pl.when(i == 0)
    def _():
        out_ref[...] = jnp.zeros((1, 1), jnp.float32)

    out_ref[...] += jnp.reshape(part, (1, 1))


@jax.jit
def kernel(logits, targets):
    n_blocks = BATCH // BR
    t3 = targets.astype(jnp.int32).reshape(n_blocks, 1, BR)
    out = pl.pallas_call(
        _loss_body,
        grid=(n_blocks,),
        in_specs=[
            pl.BlockSpec((1, 1, BR), lambda i: (i, 0, 0)),
            pl.BlockSpec(memory_space=pl.ANY),
        ],
        out_specs=pl.BlockSpec((1, 1), lambda i: (0, 0)),
        out_shape=jax.ShapeDtypeStruct((1, 1), jnp.float32),
        scratch_shapes=[
            pltpu.VMEM((2, BR, NUM_CLASSES), jnp.float32),
            pltpu.SemaphoreType.DMA((2,)),
        ],
    )(t3, logits)
    return out[0, 0]


# speedup vs baseline: 1.9633x; 1.0132x over previous
import jax
import jax.numpy as jnp
from jax import lax
from jax.experimental import pallas as pl
from jax.experimental.pallas import tpu as pltpu

NUM_CLASSES = 1000
EPS = 0.1
BATCH = 16384
A = EPS / (NUM_CLASSES - 1)
B_COEF = 1.0 - EPS - A

BR = 2048


def _loss_body(t_ref, x_hbm, out_ref, buf, sems):
    i = pl.program_id(0)
    n = pl.num_programs(0)
    slot = i % 2

    @pl.when(i == 0)
    def _():
        pltpu.make_async_copy(
            x_hbm.at[pl.ds(0, BR)], buf.at[0], sems.at[0]
        ).start()

    @pl.when(i + 1 < n)
    def _():
        pltpu.make_async_copy(
            x_hbm.at[pl.ds((i + 1) * BR, BR)],
            buf.at[(i + 1) % 2],
            sems.at[(i + 1) % 2],
        ).start()

    pltpu.make_async_copy(
        x_hbm.at[pl.ds(i * BR, BR)], buf.at[slot], sems.at[slot]
    ).wait()
    x = buf[slot]
    t = t_ref[0, 0, :]
    m = jnp.max(x, axis=1, keepdims=True)
    s = jnp.sum(jnp.exp(x - m), axis=1)
    lse = jnp.log(s) + m[:, 0]
    row_sum = jnp.sum(x, axis=1)
    col = lax.broadcasted_iota(jnp.int32, x.shape, 1)
    tgt = jnp.sum(jnp.where(col == t[:, None], x, 0.0), axis=1)
    part = jnp.sum(lse - A * row_sum - B_COEF * tgt) * (1.0 / BATCH)

    @pl.when(i == 0)
    def _():
        out_ref[...] = jnp.zeros((1, 1), jnp.float32)

    out_ref[...] += jnp.reshape(part, (1, 1))


@jax.jit
def kernel(logits, targets):
    n_blocks = BATCH // BR
    t3 = targets.astype(jnp.int32).reshape(n_blocks, 1, BR)
    out = pl.pallas_call(
        _loss_body,
        grid=(n_blocks,),
        in_specs=[
            pl.BlockSpec((1, 1, BR), lambda i: (i, 0, 0)),
            pl.BlockSpec(memory_space=pl.ANY),
        ],
        out_specs=pl.BlockSpec((1, 1), lambda i: (0, 0)),
        out_shape=jax.ShapeDtypeStruct((1, 1), jnp.float32),
        scratch_shapes=[
            pltpu.VMEM((2, BR, NUM_CLASSES), jnp.float32),
            pltpu.SemaphoreType.DMA((2,)),
        ],
        compiler_params=pltpu.CompilerParams(needs_layout_passes=False),
    )(t3, logits)
    return out[0, 0]
